# baseline (speedup 1.0000x reference)
import functools

import jax
import jax.numpy as jnp
from jax import lax
from jax.experimental import pallas as pl
from jax.experimental.pallas import tpu as pltpu
from jax.experimental.pallas import tpu_sc as plsc

N = 10000
E = 320000
D = 128
NG = 128
NC = 10
K = 5000
BLK = 1024

NPAD_C = 10240
_SC_C = 2
_SC_S = 16
_NW = _SC_C * _SC_S
_CH = 80


def _segsum_body(ew, nchunk, stripe, weighted, remap, dw, body_refs):
    it = iter(body_refs)
    table, idx_hbm, dst_hbm = next(it), next(it), next(it)
    w_hbm = next(it) if weighted else None
    zeros_hbm = next(it)
    rtab_hbm = next(it) if remap else None
    out = next(it)
    rtab_v = next(it) if remap else None
    idx_v, dst_v = next(it), next(it)
    w_v = next(it) if weighted else None
    rows_v, acc, sem = next(it), next(it), next(it)
    c = lax.axis_index("c")
    s = lax.axis_index("s")
    wid = c * _SC_S + s
    pltpu.sync_copy(zeros_hbm.at[pl.ds(0, stripe)], acc.at[pl.ds(s * stripe, stripe)])
    if remap:
        pltpu.sync_copy(rtab_hbm, rtab_v)
    plsc.subcore_barrier()

    def step(i, carry):
        base = wid * ew + i * _CH
        pltpu.sync_copy(idx_hbm.at[pl.ds(base, _CH)], idx_v)
        pltpu.sync_copy(dst_hbm.at[pl.ds(base, _CH)], dst_v)
        if remap:
            def rm(g, cc):
                sl = pl.ds(g * 16, 16)
                idx_v[sl] = plsc.load_gather(rtab_v, [idx_v[sl]])
                dst_v[sl] = plsc.load_gather(rtab_v, [dst_v[sl]])
                return cc
            lax.fori_loop(0, _CH // 16, rm, 0)
        pltpu.async_copy(table.at[idx_v], rows_v, sem).wait()
        if weighted:
            pltpu.sync_copy(w_hbm.at[pl.ds(base, _CH)], w_v)

            def scale(grp, cc):
                w16 = w_v[pl.ds(grp * 16, 16)]
                for lane in range(16):
                    e = grp * 16 + lane
                    w = w16[lane]
                    for j in range(dw // 16):
                        sl = pl.ds(j * 16, 16)
                        rows_v[e, sl] = rows_v[e, sl] * w
                return cc

            lax.fori_loop(0, _CH // 16, scale, 0)
        pltpu.sync_copy(rows_v, acc.at[dst_v], add=True)
        return carry

    lax.fori_loop(0, nchunk, step, 0)
    plsc.subcore_barrier()
    pltpu.sync_copy(acc.at[pl.ds(s * stripe, stripe)],
                    out.at[c].at[pl.ds(s * stripe, stripe)])


@functools.partial(jax.jit, static_argnames=("nseg_pad",))
def _sc_segsum(table, idx, dst, nseg_pad, weights=None, remap_tab=None):
    dw = table.shape[1]
    ew = idx.shape[0] // _NW
    nchunk = ew // _CH
    stripe = nseg_pad // _SC_S
    zeros = jnp.zeros((stripe, dw), jnp.float32)
    weighted = weights is not None
    remap = remap_tab is not None

    scratch = []
    if remap:
        scratch.append(pltpu.VMEM(remap_tab.shape, jnp.int32))
    scratch += [
        pltpu.VMEM((_CH,), jnp.int32),
        pltpu.VMEM((_CH,), jnp.int32),
    ]
    if weighted:
        scratch.append(pltpu.VMEM((_CH,), jnp.float32))
    scratch += [
        pltpu.VMEM((_CH, dw), jnp.float32),
        pltpu.VMEM_SHARED((nseg_pad, dw), jnp.float32),
        pltpu.SemaphoreType.DMA,
    ]

    def body(*refs):
        _segsum_body(ew, nchunk, stripe, weighted, remap, dw, refs)

    f = pl.kernel(
        body,
        out_type=jax.ShapeDtypeStruct((_SC_C, nseg_pad, dw), jnp.float32),
        mesh=plsc.VectorSubcoreMesh(core_axis_name="c", subcore_axis_name="s"),
        compiler_params=(pltpu.CompilerParams(needs_layout_passes=False)
                         if remap else None),
        scratch_types=scratch,
    )
    args = [table, idx, dst]
    if weighted:
        args.append(weights)
    args.append(zeros)
    if remap:
        args.append(remap_tab)
    return f(*args)


def _scalseg_body(ew, nchunk, stripe, body_refs):
    (w_hbm, dst_hbm, zeros_hbm, out, w_v, dst_v, rows_v, acc, sem) = body_refs
    c = lax.axis_index("c")
    s = lax.axis_index("s")
    wid = c * _SC_S + s
    pltpu.sync_copy(zeros_hbm.at[pl.ds(0, stripe)], acc.at[pl.ds(s * stripe, stripe)])
    pltpu.sync_copy(zeros_hbm.at[pl.ds(0, _CH)], rows_v)
    plsc.subcore_barrier()
    i0 = lax.broadcasted_iota(jnp.int32, (16,), 0)

    def step(i, carry):
        base = wid * ew + i * _CH
        pltpu.sync_copy(w_hbm.at[pl.ds(base, _CH)], w_v)
        pltpu.sync_copy(dst_hbm.at[pl.ds(base, _CH)], dst_v)

        def build(grp, cc):
            w16 = w_v[pl.ds(grp * 16, 16)]
            for lane in range(16):
                e = grp * 16 + lane
                vec = jnp.where(i0 == 0, w16[lane],
                                jnp.where(i0 == 1, 1.0, 0.0))
                rows_v[e, pl.ds(0, 16)] = vec
            return cc

        lax.fori_loop(0, _CH // 16, build, 0)
        pltpu.sync_copy(rows_v, acc.at[dst_v], add=True)
        return carry

    lax.fori_loop(0, nchunk, step, 0)
    plsc.subcore_barrier()
    pltpu.sync_copy(acc.at[pl.ds(s * stripe, stripe)],
                    out.at[c].at[pl.ds(s * stripe, stripe)])


@functools.partial(jax.jit, static_argnames=("nseg_pad",))
def _sc_scalseg(w, dstidx, nseg_pad):
    ew = w.shape[0] // _NW
    nchunk = ew // _CH
    stripe = nseg_pad // _SC_S
    zeros = jnp.zeros((stripe, D), jnp.float32)

    def body(*refs):
        _scalseg_body(ew, nchunk, stripe, refs)

    f = pl.kernel(
        body,
        out_type=jax.ShapeDtypeStruct((_SC_C, nseg_pad, D), jnp.float32),
        mesh=plsc.VectorSubcoreMesh(core_axis_name="c", subcore_axis_name="s"),
        scratch_types=[
            pltpu.VMEM((_CH,), jnp.float32),
            pltpu.VMEM((_CH,), jnp.int32),
            pltpu.VMEM((_CH, D), jnp.float32),
            pltpu.VMEM_SHARED((nseg_pad, D), jnp.float32),
            pltpu.SemaphoreType.DMA,
        ],
    )
    return f(w, dstidx, zeros)


_RS = 320
_EBUF = 11280
_SCCH = 1280


def _compact_body(*body_refs):
    (src_hbm, dst_hbm, osrc, odst, srcbuf, dstbuf, sbuf, dbuf, sem) = body_refs
    c = lax.axis_index("c")
    s = lax.axis_index("s")
    wid = c * _SC_S + s
    lo = wid * _RS
    pad_src = jnp.zeros((16,), jnp.int32)
    pad_dst = jnp.full((16,), lo + _RS, jnp.int32)

    def prefill(g, cc):
        sl = pl.ds(g * 16, 16)
        srcbuf[sl] = pad_src
        dstbuf[sl] = pad_dst
        return cc

    lax.fori_loop(0, _EBUF // 16, prefill, 0)
    limit = _EBUF - 96

    def scan_chunk(i, cnt):
        pltpu.sync_copy(src_hbm.at[pl.ds(i * _SCCH, _SCCH)], sbuf)
        pltpu.sync_copy(dst_hbm.at[pl.ds(i * _SCCH, _SCCH)], dbuf)

        def grp(g, cnt):
            sl = pl.ds(g * 16, 16)
            s16 = sbuf[sl]
            d16 = dbuf[sl]
            mask = (d16 >= lo) & (d16 < lo + _RS)
            off = jnp.minimum(cnt, limit)
            plsc.store_compressed(srcbuf.at[pl.ds(off, 16)], s16, mask=mask)
            plsc.store_compressed(dstbuf.at[pl.ds(off, 16)], d16, mask=mask)
            pc = plsc.all_reduce_population_count(mask)[0]
            return jnp.minimum(cnt + pc, limit)

        return lax.fori_loop(0, _SCCH // 16, grp, cnt)

    lax.fori_loop(0, E // _SCCH, scan_chunk, 0)
    pltpu.sync_copy(srcbuf, osrc.at[pl.ds(wid * _EBUF, _EBUF)])
    pltpu.sync_copy(dstbuf, odst.at[pl.ds(wid * _EBUF, _EBUF)])


@jax.jit
def _sc_compact(src, dst):
    f = pl.kernel(
        _compact_body,
        out_type=(jax.ShapeDtypeStruct((_NW * _EBUF,), jnp.int32),
                  jax.ShapeDtypeStruct((_NW * _EBUF,), jnp.int32)),
        mesh=plsc.VectorSubcoreMesh(core_axis_name="c", subcore_axis_name="s"),
        compiler_params=pltpu.CompilerParams(needs_layout_passes=False),
        scratch_types=[
            pltpu.VMEM((_EBUF,), jnp.int32),
            pltpu.VMEM((_EBUF,), jnp.int32),
            pltpu.VMEM((_SCCH,), jnp.int32),
            pltpu.VMEM((_SCCH,), jnp.int32),
            pltpu.SemaphoreType.DMA,
        ],
    )
    return f(src, dst)


def _rangemax_body(*body_refs):
    (table, bsrc, bdst, out, acc, rows_v, idxc, dstc, sem) = body_refs
    c = lax.axis_index("c")
    s = lax.axis_index("s")
    wid = c * _SC_S + s
    lo = wid * _RS
    pltpu.sync_copy(table.at[pl.ds(lo, _RS)], acc.at[pl.ds(0, _RS)])

    def max_chunk(i, carry):
        pltpu.sync_copy(bsrc.at[pl.ds(wid * _EBUF + i * _CH, _CH)], idxc)
        pltpu.sync_copy(bdst.at[pl.ds(wid * _EBUF + i * _CH, _CH)], dstc)
        pltpu.async_copy(table.at[idxc], rows_v, sem).wait()

        def grp(g, cc):
            d16 = dstc[pl.ds(g * 16, 16)] - lo
            for lane in range(16):
                e = g * 16 + lane
                dloc = d16[lane]
                for j in range(D // 16):
                    sl = pl.ds(j * 16, 16)
                    acc[dloc, sl] = jnp.maximum(acc[dloc, sl], rows_v[e, sl])
            return cc

        lax.fori_loop(0, _CH // 16, grp, 0)
        return carry

    lax.fori_loop(0, _EBUF // _CH, max_chunk, 0)
    pltpu.sync_copy(acc.at[pl.ds(0, _RS)], out.at[pl.ds(lo, _RS)])


@jax.jit
def _sc_segmax(table, src, dst):
    bsrc, bdst = _sc_compact(src, dst)
    f = pl.kernel(
        _rangemax_body,
        out_type=jax.ShapeDtypeStruct((NPAD_C, D), jnp.float32),
        mesh=plsc.VectorSubcoreMesh(core_axis_name="c", subcore_axis_name="s"),
        scratch_types=[
            pltpu.VMEM((_RS + 1, D), jnp.float32),
            pltpu.VMEM((_CH, D), jnp.float32),
            pltpu.VMEM((_CH,), jnp.int32),
            pltpu.VMEM((_CH,), jnp.int32),
            pltpu.SemaphoreType.DMA,
        ],
    )
    return f(table, bsrc, bdst)


def _edge_body(mode, ew, ngrp, body_refs):
    (ta_hbm, tb_hbm, ea_hbm, eb_hbm, out,
     ta_v, tb_v, ea_v, eb_v, out_v, sem) = body_refs
    c = lax.axis_index("c")
    s = lax.axis_index("s")
    wid = c * _SC_S + s
    base = wid * ew
    pltpu.sync_copy(ta_hbm, ta_v)
    pltpu.sync_copy(tb_hbm, tb_v)
    pltpu.sync_copy(ea_hbm.at[pl.ds(base, ew)], ea_v)
    pltpu.sync_copy(eb_hbm.at[pl.ds(base, ew)], eb_v)

    def grp(g, cc):
        sl = pl.ds(g * 16, 16)
        d16 = ea_v[sl]
        if mode == "score":
            s16 = eb_v[sl]
            qv = plsc.load_gather(ta_v, [d16])
            pv = plsc.load_gather(tb_v, [s16])
            t = qv + pv
            out_v[sl] = jnp.where(t > 0, t, 0.2 * t)
        else:
            scv = eb_v[sl]
            mv = plsc.load_gather(ta_v, [d16])
            out_v[sl] = jnp.exp(scv - mv)
        return cc

    lax.fori_loop(0, ngrp, grp, 0)
    pltpu.sync_copy(out_v, out.at[pl.ds(base, ew)])


@functools.partial(jax.jit, static_argnames=("mode",))
def _sc_edge(mode, ta, tb, ea, eb):
    ep = ea.shape[0]
    ew = ep // _NW
    ngrp = ew // 16
    tn = ta.shape[0]

    def body(*refs):
        _edge_body(mode, ew, ngrp, refs)

    f = pl.kernel(
        body,
        out_type=jax.ShapeDtypeStruct((ep,), jnp.float32),
        mesh=plsc.VectorSubcoreMesh(core_axis_name="c", subcore_axis_name="s"),
        compiler_params=pltpu.CompilerParams(needs_layout_passes=False),
        scratch_types=[
            pltpu.VMEM((tn,), jnp.float32),
            pltpu.VMEM((tn,), jnp.float32),
            pltpu.VMEM((ew,), jnp.int32),
            pltpu.VMEM((ew,), ea.dtype if mode == "score" else jnp.float32),
            pltpu.VMEM((ew,), jnp.float32),
            pltpu.SemaphoreType.DMA,
        ],
    )
    return f(ta, tb, ea, eb)


def _mlp_body(scale, row_limit, x_ref, a0_ref, a1_ref, w1_ref, b1_ref, w2_ref, b2_ref, o_ref):
    z = x_ref[...] + scale * (a0_ref[...] + a1_ref[...])
    z = jnp.maximum(jnp.dot(z, w1_ref[...], preferred_element_type=jnp.float32) + b1_ref[...], 0.0)
    o = jnp.maximum(jnp.dot(z, w2_ref[...], preferred_element_type=jnp.float32) + b2_ref[...], 0.0)
    if row_limit is not None:
        rid = pl.program_id(0) * BLK + jax.lax.broadcasted_iota(jnp.int32, (BLK, D), 0)
        o = jnp.where(rid < row_limit, o, 0.0)
    o_ref[...] = o


def _gin_mlp(x, a0, a1, W1, b1, W2, b2, scale=1.0, row_limit=None):
    npad = x.shape[0]
    out = pl.pallas_call(
        functools.partial(_mlp_body, scale, row_limit),
        grid=(npad // BLK,),
        in_specs=[
            pl.BlockSpec((BLK, D), lambda i: (i, 0)),
            pl.BlockSpec((BLK, D), lambda i: (i, 0)),
            pl.BlockSpec((BLK, D), lambda i: (i, 0)),
            pl.BlockSpec((D, D), lambda i: (0, 0)),
            pl.BlockSpec((1, D), lambda i: (0, 0)),
            pl.BlockSpec((D, D), lambda i: (0, 0)),
            pl.BlockSpec((1, D), lambda i: (0, 0)),
        ],
        out_specs=pl.BlockSpec((BLK, D), lambda i: (i, 0)),
        out_shape=jax.ShapeDtypeStruct((npad, D), jnp.float32),
    )(x, a0, a1, W1, b1.reshape(1, D), W2, b2.reshape(1, D))
    return out


def kernel(x, enc_W1, enc_b1, enc_W2, enc_b2, enc_W3, enc_b3, enc_W4, enc_b4,
           pool_lin_W, pool_lin_b, pool_att_W, pool_att_b,
           score_W1, score_b1, score_W2, score_W3,
           gnn_W1, gnn_b1, gnn_W2, gnn_b2, gnn_W3, gnn_b3, gnn_W4, gnn_b4,
           cls_W, cls_b, edge_index, batch):
    src, dst = edge_index[0], edge_index[1]
    NPAD = NPAD_C

    xp = jnp.pad(x, ((0, NPAD - N), (0, 0)))
    p = _sc_segsum(xp, src, dst, NPAD)
    hp = _gin_mlp(xp, p[0], p[1], enc_W1, enc_b1, enc_W2, enc_b2)
    p = _sc_segsum(hp, src, dst, NPAD)
    hp = _gin_mlp(hp, p[0], p[1], enc_W3, enc_b3, enc_W4, enc_b4)
    h = hp[:N]

    loop = jnp.arange(N, dtype=src.dtype)
    s2 = jnp.concatenate([src, loop])
    d2 = jnp.concatenate([dst, loop])
    wq = pool_att_W[:D, 0]
    wp = pool_att_W[D:, 0]
    qraw = _sc_segmax(hp, src, dst) @ (pool_lin_W @ wq)
    p_vec = h @ wp
    E2 = E + N
    EP = ((E2 + _NW * _CH - 1) // (_NW * _CH)) * (_NW * _CH)
    npadE = EP - E2
    s2p = jnp.concatenate([s2, jnp.zeros((npadE,), jnp.int32)])
    d2p = jnp.concatenate([d2, N + (jnp.arange(npadE, dtype=jnp.int32) % (NPAD - N))])
    qt = qraw + (jnp.dot(pool_lin_b, wq) + pool_att_b[0])
    pt = jnp.pad(p_vec, (0, NPAD - N))
    scp = _sc_edge("score", qt, pt, d2p, s2p)
    m = jax.ops.segment_max(scp[:E2], d2, num_segments=N)
    mt = jnp.pad(m, (0, NPAD - N))
    wpad = _sc_edge("expw", mt, mt, d2p, scp)
    xcd = _sc_segsum(hp, s2p, d2p, NPAD, weights=wpad)
    dc = _sc_scalseg(wpad, d2p, NPAD)
    den = dc[0, :N, 0] + dc[1, :N, 0]
    cnt_in = dc[0, :N, 1] + dc[1, :N, 1]
    xc = (xcd[0, :N] + xcd[1, :N]) / (den + 1e-16)[:, None]

    xce = jnp.pad(xc, ((0, NPAD - N), (0, 0)))
    rs = _sc_segsum(xce, s2p, d2p, NPAD)
    rsum = rs[0, :N] + rs[1, :N]
    a1 = (xc @ score_W1 + score_b1)[:, 0]
    a2 = (xc @ score_W2)[:, 0]
    t3s = (rsum @ score_W3)[:, 0]
    fit = a1 + cnt_in * a2 - t3s
    fitness = jax.nn.sigmoid(fit)
    topv, perm = jax.lax.top_k(fitness, K)
    px = xc[perm] * topv[:, None]
    pbatch = batch[perm]
    kept = jnp.zeros((N,), jnp.float32).at[perm].set(1.0)
    nid = jnp.zeros((N,), src.dtype).at[perm].set(jnp.arange(K, dtype=src.dtype))
    KPAD = 5120
    KTAB = 13696
    SIG1 = 0.7310585786300049
    nid_ext = jnp.where(kept > 0.5, nid,
                        KPAD + jnp.arange(N, dtype=jnp.int32) % (KTAB - KPAD))
    nid_ext = jnp.pad(nid_ext, (0, NPAD - N))
    pxz = jnp.pad(px, ((0, KTAB - K), (0, 0)))
    p = _sc_segsum(pxz, src, dst, KTAB, remap_tab=nid_ext)
    g = _gin_mlp(pxz[:KPAD], p[0, :KPAD], p[1, :KPAD], gnn_W1, gnn_b1,
                 gnn_W2, gnn_b2, scale=SIG1, row_limit=K)
    gz = jnp.pad(g, ((0, KTAB - KPAD), (0, 0)))
    p = _sc_segsum(gz, src, dst, KTAB, remap_tab=nid_ext)
    g = _gin_mlp(g, p[0, :KPAD], p[1, :KPAD], gnn_W3, gnn_b3, gnn_W4, gnn_b4,
                 scale=SIG1, row_limit=K)
    g = g[:K]

    sums = jax.ops.segment_sum(g, pbatch, num_segments=NG)
    cnt = jax.ops.segment_sum(jnp.ones((K,), jnp.float32), pbatch, num_segments=NG)
    readout = sums / jnp.maximum(cnt, 1.0)[:, None]
    return readout @ cls_W + cls_b


# speedup vs baseline: 1.0661x; 1.0661x over previous
import functools

import jax
import jax.numpy as jnp
from jax import lax
from jax.experimental import pallas as pl
from jax.experimental.pallas import tpu as pltpu
from jax.experimental.pallas import tpu_sc as plsc

N = 10000
E = 320000
D = 128
NG = 128
NC = 10
K = 5000
BLK = 1024

NPAD_C = 10240
_SC_C = 2
_SC_S = 16
_NW = _SC_C * _SC_S
_CH = 80


def _segsum_body(ew, nchunk, stripe, weighted, remap, dw, body_refs):
    it = iter(body_refs)
    table, idx_hbm, dst_hbm = next(it), next(it), next(it)
    w_hbm = next(it) if weighted else None
    zeros_hbm = next(it)
    rtab_hbm = next(it) if remap else None
    out = next(it)
    rtab_v = next(it) if remap else None
    idx_v, dst_v = next(it), next(it)
    w_v = next(it) if weighted else None
    rows_v, acc, sem = next(it), next(it), next(it)
    c = lax.axis_index("c")
    s = lax.axis_index("s")
    wid = c * _SC_S + s
    pltpu.sync_copy(zeros_hbm.at[pl.ds(0, stripe)], acc.at[pl.ds(s * stripe, stripe)])
    if remap:
        pltpu.sync_copy(rtab_hbm, rtab_v)
    plsc.subcore_barrier()

    def step(i, carry):
        base = wid * ew + i * _CH
        pltpu.sync_copy(idx_hbm.at[pl.ds(base, _CH)], idx_v)
        pltpu.sync_copy(dst_hbm.at[pl.ds(base, _CH)], dst_v)
        if remap:
            def rm(g, cc):
                sl = pl.ds(g * 16, 16)
                idx_v[sl] = plsc.load_gather(rtab_v, [idx_v[sl]])
                dst_v[sl] = plsc.load_gather(rtab_v, [dst_v[sl]])
                return cc
            lax.fori_loop(0, _CH // 16, rm, 0)
        pltpu.async_copy(table.at[idx_v], rows_v, sem).wait()
        if weighted:
            pltpu.sync_copy(w_hbm.at[pl.ds(base, _CH)], w_v)

            def scale(grp, cc):
                w16 = w_v[pl.ds(grp * 16, 16)]
                for lane in range(16):
                    e = grp * 16 + lane
                    w = w16[lane]
                    for j in range(dw // 16):
                        sl = pl.ds(j * 16, 16)
                        rows_v[e, sl] = rows_v[e, sl] * w
                return cc

            lax.fori_loop(0, _CH // 16, scale, 0)
        pltpu.sync_copy(rows_v, acc.at[dst_v], add=True)
        return carry

    lax.fori_loop(0, nchunk, step, 0)
    plsc.subcore_barrier()
    pltpu.sync_copy(acc.at[pl.ds(s * stripe, stripe)],
                    out.at[c].at[pl.ds(s * stripe, stripe)])


@functools.partial(jax.jit, static_argnames=("nseg_pad",))
def _sc_segsum(table, idx, dst, nseg_pad, weights=None, remap_tab=None):
    dw = table.shape[1]
    ew = idx.shape[0] // _NW
    nchunk = ew // _CH
    stripe = nseg_pad // _SC_S
    zeros = jnp.zeros((stripe, dw), jnp.float32)
    weighted = weights is not None
    remap = remap_tab is not None

    scratch = []
    if remap:
        scratch.append(pltpu.VMEM(remap_tab.shape, jnp.int32))
    scratch += [
        pltpu.VMEM((_CH,), jnp.int32),
        pltpu.VMEM((_CH,), jnp.int32),
    ]
    if weighted:
        scratch.append(pltpu.VMEM((_CH,), jnp.float32))
    scratch += [
        pltpu.VMEM((_CH, dw), jnp.float32),
        pltpu.VMEM_SHARED((nseg_pad, dw), jnp.float32),
        pltpu.SemaphoreType.DMA,
    ]

    def body(*refs):
        _segsum_body(ew, nchunk, stripe, weighted, remap, dw, refs)

    f = pl.kernel(
        body,
        out_type=jax.ShapeDtypeStruct((_SC_C, nseg_pad, dw), jnp.float32),
        mesh=plsc.VectorSubcoreMesh(core_axis_name="c", subcore_axis_name="s"),
        compiler_params=(pltpu.CompilerParams(needs_layout_passes=False)
                         if remap else None),
        scratch_types=scratch,
    )
    args = [table, idx, dst]
    if weighted:
        args.append(weights)
    args.append(zeros)
    if remap:
        args.append(remap_tab)
    return f(*args)


def _scalseg_body(ew, nchunk, stripe, body_refs):
    (w_hbm, dst_hbm, zeros_hbm, out, w_v, dst_v, rows_v, acc, sem) = body_refs
    c = lax.axis_index("c")
    s = lax.axis_index("s")
    wid = c * _SC_S + s
    pltpu.sync_copy(zeros_hbm.at[pl.ds(0, stripe)], acc.at[pl.ds(s * stripe, stripe)])
    pltpu.sync_copy(zeros_hbm.at[pl.ds(0, _CH)], rows_v)
    plsc.subcore_barrier()
    i0 = lax.broadcasted_iota(jnp.int32, (16,), 0)

    def step(i, carry):
        base = wid * ew + i * _CH
        pltpu.sync_copy(w_hbm.at[pl.ds(base, _CH)], w_v)
        pltpu.sync_copy(dst_hbm.at[pl.ds(base, _CH)], dst_v)

        def build(grp, cc):
            w16 = w_v[pl.ds(grp * 16, 16)]
            for lane in range(16):
                e = grp * 16 + lane
                vec = jnp.where(i0 == 0, w16[lane],
                                jnp.where(i0 == 1, 1.0, 0.0))
                rows_v[e, pl.ds(0, 16)] = vec
            return cc

        lax.fori_loop(0, _CH // 16, build, 0)
        pltpu.sync_copy(rows_v, acc.at[dst_v], add=True)
        return carry

    lax.fori_loop(0, nchunk, step, 0)
    plsc.subcore_barrier()
    pltpu.sync_copy(acc.at[pl.ds(s * stripe, stripe)],
                    out.at[c].at[pl.ds(s * stripe, stripe)])


@functools.partial(jax.jit, static_argnames=("nseg_pad",))
def _sc_scalseg(w, dstidx, nseg_pad):
    ew = w.shape[0] // _NW
    nchunk = ew // _CH
    stripe = nseg_pad // _SC_S
    zeros = jnp.zeros((stripe, D), jnp.float32)

    def body(*refs):
        _scalseg_body(ew, nchunk, stripe, refs)

    f = pl.kernel(
        body,
        out_type=jax.ShapeDtypeStruct((_SC_C, nseg_pad, D), jnp.float32),
        mesh=plsc.VectorSubcoreMesh(core_axis_name="c", subcore_axis_name="s"),
        scratch_types=[
            pltpu.VMEM((_CH,), jnp.float32),
            pltpu.VMEM((_CH,), jnp.int32),
            pltpu.VMEM((_CH, D), jnp.float32),
            pltpu.VMEM_SHARED((nseg_pad, D), jnp.float32),
            pltpu.SemaphoreType.DMA,
        ],
    )
    return f(w, dstidx, zeros)


_RS = 320
_EBUF = 11280
_SCCH = 1280


def _compact_body(*body_refs):
    (src_hbm, dst_hbm, osrc, odst, srcbuf, dstbuf, sbuf, dbuf, sem) = body_refs
    c = lax.axis_index("c")
    s = lax.axis_index("s")
    wid = c * _SC_S + s
    lo = wid * _RS
    pad_src = jnp.zeros((16,), jnp.int32)
    pad_dst = jnp.full((16,), lo + _RS, jnp.int32)

    def prefill(g, cc):
        sl = pl.ds(g * 16, 16)
        srcbuf[sl] = pad_src
        dstbuf[sl] = pad_dst
        return cc

    lax.fori_loop(0, _EBUF // 16, prefill, 0)
    limit = _EBUF - 96

    def scan_chunk(i, cnt):
        pltpu.sync_copy(src_hbm.at[pl.ds(i * _SCCH, _SCCH)], sbuf)
        pltpu.sync_copy(dst_hbm.at[pl.ds(i * _SCCH, _SCCH)], dbuf)

        def grp(g, cnt):
            sl = pl.ds(g * 16, 16)
            s16 = sbuf[sl]
            d16 = dbuf[sl]
            mask = (d16 >= lo) & (d16 < lo + _RS)
            off = jnp.minimum(cnt, limit)
            plsc.store_compressed(srcbuf.at[pl.ds(off, 16)], s16, mask=mask)
            plsc.store_compressed(dstbuf.at[pl.ds(off, 16)], d16, mask=mask)
            pc = plsc.all_reduce_population_count(mask)[0]
            return jnp.minimum(cnt + pc, limit)

        return lax.fori_loop(0, _SCCH // 16, grp, cnt)

    lax.fori_loop(0, E // _SCCH, scan_chunk, 0)
    pltpu.sync_copy(srcbuf, osrc.at[pl.ds(wid * _EBUF, _EBUF)])
    pltpu.sync_copy(dstbuf, odst.at[pl.ds(wid * _EBUF, _EBUF)])


@jax.jit
def _sc_compact(src, dst):
    f = pl.kernel(
        _compact_body,
        out_type=(jax.ShapeDtypeStruct((_NW * _EBUF,), jnp.int32),
                  jax.ShapeDtypeStruct((_NW * _EBUF,), jnp.int32)),
        mesh=plsc.VectorSubcoreMesh(core_axis_name="c", subcore_axis_name="s"),
        compiler_params=pltpu.CompilerParams(needs_layout_passes=False),
        scratch_types=[
            pltpu.VMEM((_EBUF,), jnp.int32),
            pltpu.VMEM((_EBUF,), jnp.int32),
            pltpu.VMEM((_SCCH,), jnp.int32),
            pltpu.VMEM((_SCCH,), jnp.int32),
            pltpu.SemaphoreType.DMA,
        ],
    )
    return f(src, dst)


def _rangemax_body(*body_refs):
    (table, bsrc, bdst, out, acc, rows_v, idxc, dstc, sem) = body_refs
    c = lax.axis_index("c")
    s = lax.axis_index("s")
    wid = c * _SC_S + s
    lo = wid * _RS
    pltpu.sync_copy(table.at[pl.ds(lo, _RS)], acc.at[pl.ds(0, _RS)])

    def max_chunk(i, carry):
        pltpu.sync_copy(bsrc.at[pl.ds(wid * _EBUF + i * _CH, _CH)], idxc)
        pltpu.sync_copy(bdst.at[pl.ds(wid * _EBUF + i * _CH, _CH)], dstc)
        pltpu.async_copy(table.at[idxc], rows_v, sem).wait()

        def grp(g, cc):
            d16 = dstc[pl.ds(g * 16, 16)] - lo
            for lane in range(16):
                e = g * 16 + lane
                dloc = d16[lane]
                for j in range(D // 16):
                    sl = pl.ds(j * 16, 16)
                    acc[dloc, sl] = jnp.maximum(acc[dloc, sl], rows_v[e, sl])
            return cc

        lax.fori_loop(0, _CH // 16, grp, 0)
        return carry

    lax.fori_loop(0, _EBUF // _CH, max_chunk, 0)
    pltpu.sync_copy(acc.at[pl.ds(0, _RS)], out.at[pl.ds(lo, _RS)])


@jax.jit
def _sc_segmax(table, src, dst):
    bsrc, bdst = _sc_compact(src, dst)
    f = pl.kernel(
        _rangemax_body,
        out_type=jax.ShapeDtypeStruct((NPAD_C, D), jnp.float32),
        mesh=plsc.VectorSubcoreMesh(core_axis_name="c", subcore_axis_name="s"),
        scratch_types=[
            pltpu.VMEM((_RS + 1, D), jnp.float32),
            pltpu.VMEM((_CH, D), jnp.float32),
            pltpu.VMEM((_CH,), jnp.int32),
            pltpu.VMEM((_CH,), jnp.int32),
            pltpu.SemaphoreType.DMA,
        ],
    )
    return f(table, bsrc, bdst)


def _edge_body(mode, ew, ngrp, body_refs):
    (ta_hbm, tb_hbm, ea_hbm, eb_hbm, out,
     ta_v, tb_v, ea_v, eb_v, out_v, sem) = body_refs
    c = lax.axis_index("c")
    s = lax.axis_index("s")
    wid = c * _SC_S + s
    base = wid * ew
    pltpu.sync_copy(ta_hbm, ta_v)
    pltpu.sync_copy(tb_hbm, tb_v)
    pltpu.sync_copy(ea_hbm.at[pl.ds(base, ew)], ea_v)
    pltpu.sync_copy(eb_hbm.at[pl.ds(base, ew)], eb_v)

    def grp(g, cc):
        sl = pl.ds(g * 16, 16)
        d16 = ea_v[sl]
        if mode == "score":
            s16 = eb_v[sl]
            qv = plsc.load_gather(ta_v, [d16])
            pv = plsc.load_gather(tb_v, [s16])
            t = qv + pv
            out_v[sl] = jnp.where(t > 0, t, 0.2 * t)
        else:
            scv = eb_v[sl]
            mv = plsc.load_gather(ta_v, [d16])
            out_v[sl] = jnp.exp(scv - mv)
        return cc

    lax.fori_loop(0, ngrp, grp, 0)
    pltpu.sync_copy(out_v, out.at[pl.ds(base, ew)])


@functools.partial(jax.jit, static_argnames=("mode",))
def _sc_edge(mode, ta, tb, ea, eb):
    ep = ea.shape[0]
    ew = ep // _NW
    ngrp = ew // 16
    tn = ta.shape[0]

    def body(*refs):
        _edge_body(mode, ew, ngrp, refs)

    f = pl.kernel(
        body,
        out_type=jax.ShapeDtypeStruct((ep,), jnp.float32),
        mesh=plsc.VectorSubcoreMesh(core_axis_name="c", subcore_axis_name="s"),
        compiler_params=pltpu.CompilerParams(needs_layout_passes=False),
        scratch_types=[
            pltpu.VMEM((tn,), jnp.float32),
            pltpu.VMEM((tn,), jnp.float32),
            pltpu.VMEM((ew,), jnp.int32),
            pltpu.VMEM((ew,), ea.dtype if mode == "score" else jnp.float32),
            pltpu.VMEM((ew,), jnp.float32),
            pltpu.SemaphoreType.DMA,
        ],
    )
    return f(ta, tb, ea, eb)


def _mlp_body(scale, row_limit, x_ref, a0_ref, a1_ref, w1_ref, b1_ref, w2_ref, b2_ref, o_ref):
    z = x_ref[...] + scale * (a0_ref[...] + a1_ref[...])
    z = jnp.maximum(jnp.dot(z, w1_ref[...], preferred_element_type=jnp.float32) + b1_ref[...], 0.0)
    o = jnp.maximum(jnp.dot(z, w2_ref[...], preferred_element_type=jnp.float32) + b2_ref[...], 0.0)
    if row_limit is not None:
        rid = pl.program_id(0) * BLK + jax.lax.broadcasted_iota(jnp.int32, (BLK, D), 0)
        o = jnp.where(rid < row_limit, o, 0.0)
    o_ref[...] = o


def _gin_mlp(x, a0, a1, W1, b1, W2, b2, scale=1.0, row_limit=None):
    npad = x.shape[0]
    out = pl.pallas_call(
        functools.partial(_mlp_body, scale, row_limit),
        grid=(npad // BLK,),
        in_specs=[
            pl.BlockSpec((BLK, D), lambda i: (i, 0)),
            pl.BlockSpec((BLK, D), lambda i: (i, 0)),
            pl.BlockSpec((BLK, D), lambda i: (i, 0)),
            pl.BlockSpec((D, D), lambda i: (0, 0)),
            pl.BlockSpec((1, D), lambda i: (0, 0)),
            pl.BlockSpec((D, D), lambda i: (0, 0)),
            pl.BlockSpec((1, D), lambda i: (0, 0)),
        ],
        out_specs=pl.BlockSpec((BLK, D), lambda i: (i, 0)),
        out_shape=jax.ShapeDtypeStruct((npad, D), jnp.float32),
    )(x, a0, a1, W1, b1.reshape(1, D), W2, b2.reshape(1, D))
    return out


def kernel(x, enc_W1, enc_b1, enc_W2, enc_b2, enc_W3, enc_b3, enc_W4, enc_b4,
           pool_lin_W, pool_lin_b, pool_att_W, pool_att_b,
           score_W1, score_b1, score_W2, score_W3,
           gnn_W1, gnn_b1, gnn_W2, gnn_b2, gnn_W3, gnn_b3, gnn_W4, gnn_b4,
           cls_W, cls_b, edge_index, batch):
    src, dst = edge_index[0], edge_index[1]
    NPAD = NPAD_C

    xp = jnp.pad(x, ((0, NPAD - N), (0, 0)))
    p = _sc_segsum(xp, src, dst, NPAD)
    hp = _gin_mlp(xp, p[0], p[1], enc_W1, enc_b1, enc_W2, enc_b2)
    p = _sc_segsum(hp, src, dst, NPAD)
    hp = _gin_mlp(hp, p[0], p[1], enc_W3, enc_b3, enc_W4, enc_b4)
    h = hp[:N]

    loop = jnp.arange(N, dtype=src.dtype)
    s2 = jnp.concatenate([src, loop])
    d2 = jnp.concatenate([dst, loop])
    wq = pool_att_W[:D, 0]
    wp = pool_att_W[D:, 0]
    xq = jax.ops.segment_max(h[s2], d2, num_segments=N)
    qraw = jnp.pad(xq @ (pool_lin_W @ wq), (0, NPAD - N))
    p_vec = h @ wp
    E2 = E + N
    EP = ((E2 + _NW * _CH - 1) // (_NW * _CH)) * (_NW * _CH)
    npadE = EP - E2
    s2p = jnp.concatenate([s2, jnp.zeros((npadE,), jnp.int32)])
    d2p = jnp.concatenate([d2, N + (jnp.arange(npadE, dtype=jnp.int32) % (NPAD - N))])
    qt = qraw + (jnp.dot(pool_lin_b, wq) + pool_att_b[0])
    qt = qt.at[N:].set(0.0)
    pt = jnp.pad(p_vec, (0, NPAD - N))
    scp = _sc_edge("score", qt, pt, d2p, s2p)
    m = jax.ops.segment_max(scp[:E2], d2, num_segments=N)
    mt = jnp.pad(m, (0, NPAD - N))
    wpad = _sc_edge("expw", mt, mt, d2p, scp)
    xcd = _sc_segsum(hp, s2p, d2p, NPAD, weights=wpad)
    dc = _sc_scalseg(wpad, d2p, NPAD)
    den = dc[0, :N, 0] + dc[1, :N, 0]
    cnt_in = dc[0, :N, 1] + dc[1, :N, 1]
    xc = (xcd[0, :N] + xcd[1, :N]) / (den + 1e-16)[:, None]

    xce = jnp.pad(xc, ((0, NPAD - N), (0, 0)))
    rs = _sc_segsum(xce, s2p, d2p, NPAD)
    rsum = rs[0, :N] + rs[1, :N]
    a1 = (xc @ score_W1 + score_b1)[:, 0]
    a2 = (xc @ score_W2)[:, 0]
    t3s = (rsum @ score_W3)[:, 0]
    fit = a1 + cnt_in * a2 - t3s
    fitness = jax.nn.sigmoid(fit)
    topv, perm = jax.lax.top_k(fitness, K)
    px = xc[perm] * topv[:, None]
    pbatch = batch[perm]
    kept = jnp.zeros((N,), jnp.float32).at[perm].set(1.0)
    nid = jnp.zeros((N,), src.dtype).at[perm].set(jnp.arange(K, dtype=src.dtype))
    KPAD = 5120
    KTAB = 13696
    SIG1 = 0.7310585786300049
    nid_ext = jnp.where(kept > 0.5, nid,
                        KPAD + jnp.arange(N, dtype=jnp.int32) % (KTAB - KPAD))
    nid_ext = jnp.pad(nid_ext, (0, NPAD - N))
    pxz = jnp.pad(px, ((0, KTAB - K), (0, 0)))
    p = _sc_segsum(pxz, src, dst, KTAB, remap_tab=nid_ext)
    g = _gin_mlp(pxz[:KPAD], p[0, :KPAD], p[1, :KPAD], gnn_W1, gnn_b1,
                 gnn_W2, gnn_b2, scale=SIG1, row_limit=K)
    gz = jnp.pad(g, ((0, KTAB - KPAD), (0, 0)))
    p = _sc_segsum(gz, src, dst, KTAB, remap_tab=nid_ext)
    g = _gin_mlp(g, p[0, :KPAD], p[1, :KPAD], gnn_W3, gnn_b3, gnn_W4, gnn_b4,
                 scale=SIG1, row_limit=K)
    g = g[:K]

    sums = jax.ops.segment_sum(g, pbatch, num_segments=NG)
    cnt = jax.ops.segment_sum(jnp.ones((K,), jnp.float32), pbatch, num_segments=NG)
    readout = sums / jnp.maximum(cnt, 1.0)[:, None]
    return readout @ cls_W + cls_b
